# Initial kernel scaffold; baseline (speedup 1.0000x reference)
import jax
import jax.numpy as jnp
from jax.experimental import pallas as pl


def kernel(x, gamma, beta):
    raise NotImplementedError("write your pallas kernel here")


# speedup vs baseline: 1.8875x; 1.8875x over previous
import numpy as np

import jax
import jax.numpy as jnp
from jax.experimental import pallas as pl
from jax.experimental.pallas import tpu as pltpu

_EPS = 1e-5
_VMEM_LIMIT = 56 * 1024 * 1024


def _largest_divisor(n, cap):
    for t in range(min(cap, n), 0, -1):
        if n % t == 0:
            return t
    return 1


def _stats_kernel(x_ref, s_ref, q_ref):
    _, cb, d, h, w = x_ref.shape
    x = x_ref[...].reshape(cb, d * h, w)
    s_ref[...] = jnp.sum(x, axis=(1, 2)).reshape(cb, 1)
    q_ref[...] = jnp.sum(x * x, axis=(1, 2)).reshape(cb, 1)


def _up_kernel(a_ref, b_ref, x_ref, rep_ref, o_ref):
    _, cb, d, h, w = x_ref.shape
    a = a_ref[...][:, :, None]
    b = b_ref[...][:, :, None]
    xf = x_ref[...].reshape(cb * d, h, w)
    y = (xf * a + b).astype(jnp.bfloat16)
    z = jnp.dot(y.reshape(cb * d * h, w), rep_ref[...],
                preferred_element_type=jnp.float32)
    z = z.reshape(cb, d, h, 2 * w)
    for pd in range(2):
        for ph in range(2):
            o_ref[0, :, :, pd, pl.Slice(ph, h, 2), :] = z


def _channel_stats(x):
    N, C, D, H, W = x.shape
    cb = _largest_divisor(C, max(1, 2 * 1024 * 1024 // (D * H * W * 4)))
    nblk = C // cb
    s, q = pl.pallas_call(
        _stats_kernel,
        out_shape=(jax.ShapeDtypeStruct((N * C, 1), jnp.float32),
                   jax.ShapeDtypeStruct((N * C, 1), jnp.float32)),
        grid=(N * nblk,),
        in_specs=[pl.BlockSpec((1, cb, D, H, W),
                               lambda i, nb=nblk: (i // nb, i % nb, 0, 0, 0))],
        out_specs=(pl.BlockSpec((cb, 1), lambda i: (i, 0)),
                   pl.BlockSpec((cb, 1), lambda i: (i, 0))),
        compiler_params=pltpu.CompilerParams(
            dimension_semantics=("parallel",),
            vmem_limit_bytes=_VMEM_LIMIT),
    )(x)
    return s, q


def kernel(x, gamma, beta):
    N, C, D, H, W = x.shape
    R = N * C * D

    s, q = _channel_stats(x)
    cnt = jnp.float32(N * D * H * W)
    mean = s.reshape(N, C).sum(axis=0) / cnt
    ex2 = q.reshape(N, C).sum(axis=0) / cnt
    var = jnp.maximum(ex2 - mean * mean, 0.0)
    a_c = gamma.astype(jnp.float32) * jax.lax.rsqrt(var + _EPS)
    b_c = beta.astype(jnp.float32) - mean * a_c
    a_r = jnp.tile(jnp.repeat(a_c, D), N).reshape(R, 1)
    b_r = jnp.tile(jnp.repeat(b_c, D), N).reshape(R, 1)

    u = np.arange(W)[:, None]
    v = np.arange(2 * W)[None, :]
    rep = jnp.asarray(v // 2 == u, dtype=jnp.bfloat16)

    cb = _largest_divisor(C, max(1, 4 * 1024 * 1024 // (D * 2 * 2 * H * W * 4)))
    nblk = C // cb

    y = pl.pallas_call(
        _up_kernel,
        out_shape=jax.ShapeDtypeStruct((N, C, D, 2, 2 * H, 2 * W), jnp.float32),
        grid=(N * nblk,),
        in_specs=[
            pl.BlockSpec((cb * D, 1), lambda i: (i, 0)),
            pl.BlockSpec((cb * D, 1), lambda i: (i, 0)),
            pl.BlockSpec((1, cb, D, H, W),
                         lambda i, nb=nblk: (i // nb, i % nb, 0, 0, 0)),
            pl.BlockSpec((W, 2 * W), lambda i: (0, 0)),
        ],
        out_specs=pl.BlockSpec((1, cb, D, 2, 2 * H, 2 * W),
                               lambda i, nb=nblk: (i // nb, i % nb, 0, 0, 0, 0)),
        compiler_params=pltpu.CompilerParams(
            dimension_semantics=("parallel",),
            vmem_limit_bytes=_VMEM_LIMIT),
    )(a_r, b_r, x, rep)

    return y.reshape(N, C, 2 * D, 2 * H, 2 * W)
